# baseline (speedup 1.0000x reference)
import functools

import jax
import jax.numpy as jnp
from jax import lax
from jax.experimental import pallas as pl
from jax.experimental.pallas import tpu as pltpu
from jax.experimental.pallas import tpu_sc as plsc

E = 8
K = 2
D = 1024
F = 4096
N = 2048
BLK = 128
R = N * K + E * BLK
NT = R // BLK
FBLK = 512
NF = F // FBLK

_f32 = jnp.float32
_i32 = jnp.int32


def _router_body(x_ref, wr_ref, pos_ref, prob_ref, eid_ref):
    x = x_ref[...]
    wr = wr_ref[...]
    logits = lax.dot_general(x, wr, (((1,), (1,)), ((), ())),
                             preferred_element_type=_f32)
    col = lax.broadcasted_iota(_i32, (N, E), 1)
    l0 = jnp.max(logits, axis=1, keepdims=True)
    e0 = jnp.min(jnp.where(logits == l0, col, E), axis=1, keepdims=True)
    oh0 = (col == e0)
    masked = jnp.where(oh0, -jnp.inf, logits)
    l1 = jnp.max(masked, axis=1, keepdims=True)
    e1 = jnp.min(jnp.where(masked == l1, col, E), axis=1, keepdims=True)
    oh1 = (col == e1)
    z = jnp.exp(l1 - l0)
    p0 = 1.0 / (1.0 + z)
    p1 = z / (1.0 + z)
    row_i = lax.broadcasted_iota(_i32, (N, N), 0)
    col_i = lax.broadcasted_iota(_i32, (N, N), 1)
    lt = (col_i <= row_i).astype(_f32)
    oh0f = oh0.astype(_f32)
    oh1f = oh1.astype(_f32)
    c0 = lax.dot_general(lt, oh0f, (((1,), (0,)), ((), ())),
                         preferred_element_type=_f32)
    c1 = lax.dot_general(lt, oh1f, (((1,), (0,)), ((), ())),
                         preferred_element_type=_f32)
    g0 = jnp.sum(oh0f, axis=0, keepdims=True)
    g1 = jnp.sum(oh1f, axis=0, keepdims=True)
    g = (g0 + g1).astype(_i32)
    gp = ((g + (BLK - 1)) // BLK) * BLK
    gpf = gp.astype(_f32)
    er = lax.broadcasted_iota(_i32, (E, E), 0)
    ec = lax.broadcasted_iota(_i32, (E, E), 1)
    lts = (er < ec).astype(_f32)
    po = lax.dot_general(gpf, lts, (((1,), (0,)), ((), ())),
                         preferred_element_type=_f32)
    rank0 = jnp.sum(c0 * oh0f, axis=1, keepdims=True) - 1.0
    pos0 = jnp.sum(oh0f * po, axis=1, keepdims=True) + rank0
    pos1 = (jnp.sum(oh1f * (po + g0 + c1), axis=1, keepdims=True) - 1.0)
    pos_ref[0, :] = pos0[:, 0].astype(_i32)
    pos_ref[1, :] = pos1[:, 0].astype(_i32)
    prob_ref[0, :] = p0[:, 0]
    prob_ref[1, :] = p1[:, 0]
    upper = (po + gpf).astype(_i32)
    tpos = lax.broadcasted_iota(_i32, (NT, E), 0) * BLK
    eid = jnp.sum((tpos >= upper).astype(_i32), axis=1)
    eid_ref[0, :] = jnp.minimum(eid, E - 1)


def _router(x_flat, wr, interpret=False):
    return pl.pallas_call(
        _router_body,
        out_shape=(
            jax.ShapeDtypeStruct((K, N), _i32),
            jax.ShapeDtypeStruct((K, N), _f32),
            jax.ShapeDtypeStruct((1, NT), _i32),
        ),
        interpret=interpret,
    )(x_flat, wr)


_bf16 = jnp.bfloat16


def _mlp_body(eid_ref, x_ref, w1_ref, w2_ref, y_ref, w1b_ref, w2b_ref):
    f = pl.program_id(0)
    t = pl.program_id(1)
    fresh = jnp.logical_or(t == 0, eid_ref[t] != eid_ref[jnp.maximum(t - 1, 0)])

    @pl.when(fresh)
    def _():
        w1b_ref[...] = w1_ref[0].astype(_bf16)
        w2b_ref[...] = w2_ref[0].astype(_bf16)

    rows = pl.ds(t * BLK, BLK)
    xt = x_ref[rows, :].astype(_bf16)
    h = lax.dot_general(xt, w1b_ref[...], (((1,), (1,)), ((), ())),
                        preferred_element_type=_f32)
    h = jnp.square(jnp.maximum(h, 0.0)).astype(_bf16)
    y = lax.dot_general(h, w2b_ref[...], (((1,), (1,)), ((), ())),
                        preferred_element_type=_f32)

    @pl.when(f == 0)
    def _():
        y_ref[rows, :] = y

    @pl.when(f > 0)
    def _():
        y_ref[rows, :] += y


def _grouped_mlp(x_sorted, w1, w2, tile_eid, interpret=False):
    grid_spec = pltpu.PrefetchScalarGridSpec(
        num_scalar_prefetch=1,
        grid=(NF, NT),
        in_specs=[
            pl.BlockSpec((R, D), lambda f, t, eid: (0, 0)),
            pl.BlockSpec((1, FBLK, D), lambda f, t, eid: (eid[t], f, 0)),
            pl.BlockSpec((1, D, FBLK), lambda f, t, eid: (eid[t], 0, f)),
        ],
        out_specs=pl.BlockSpec((R, D), lambda f, t, eid: (0, 0)),
        scratch_shapes=[
            pltpu.VMEM((FBLK, D), _bf16),
            pltpu.VMEM((D, FBLK), _bf16),
        ],
    )
    return pl.pallas_call(
        _mlp_body,
        grid_spec=grid_spec,
        out_shape=jax.ShapeDtypeStruct((R, D), _f32),
        compiler_params=pltpu.CompilerParams(
            dimension_semantics=("arbitrary", "arbitrary")),
        interpret=interpret,
    )(tile_eid, x_sorted, w1, w2)


NC, NS = 2, 16
NW = NC * NS
A_PER_W = (N * K) // NW
CH = 32
NCH = A_PER_W // CH


def _dispatch_body(x_hbm, pos_hbm, xs_hbm, idx_v, bufs, sem, osem0, osem1):
    wid = lax.axis_index("s") * NC + lax.axis_index("c")
    base = wid * A_PER_W
    src0 = base % N
    pltpu.sync_copy(pos_hbm.at[wid], idx_v)
    osems = [osem0, osem1]
    pending = [None, None]
    for j in range(NCH):
        b = j % 2
        if pending[b] is not None:
            pending[b].wait()
        pltpu.async_copy(x_hbm.at[pl.ds(src0 + j * CH, CH)], bufs.at[b],
                         sem).wait()
        pending[b] = pltpu.async_copy(bufs.at[b], xs_hbm.at[idx_v.at[j]],
                                      osems[b])
    for b in range(2):
        if pending[b] is not None:
            pending[b].wait()


def _dispatch(x_flat, pos3):
    mesh = plsc.VectorSubcoreMesh(core_axis_name="c", subcore_axis_name="s")
    f = functools.partial(
        pl.kernel,
        out_type=jax.ShapeDtypeStruct((R, D), _f32),
        mesh=mesh,
        scratch_types=[
            pltpu.VMEM((NCH, CH), _i32),
            pltpu.VMEM((2, CH, D), _f32),
            pltpu.SemaphoreType.DMA,
            pltpu.SemaphoreType.DMA,
            pltpu.SemaphoreType.DMA,
        ],
    )(_dispatch_body)
    return f(x_flat, pos3)


T_PER_W = N // NW
TCH = 32
NTCH = T_PER_W // TCH


def _combine_body(y_hbm, pos_hbm, prob_hbm, out_hbm,
                  idx0, idx1, p0v, p1v, buf0, buf1, obuf, sem0, sem1):
    wid = lax.axis_index("s") * NC + lax.axis_index("c")
    for c in range(NTCH):
        tok = wid * T_PER_W + c * TCH
        pltpu.sync_copy(pos_hbm.at[0, pl.ds(tok, TCH)], idx0)
        pltpu.sync_copy(pos_hbm.at[1, pl.ds(tok, TCH)], idx1)
        pltpu.sync_copy(prob_hbm.at[0, pl.ds(tok, TCH)], p0v)
        pltpu.sync_copy(prob_hbm.at[1, pl.ds(tok, TCH)], p1v)
        g0 = pltpu.async_copy(y_hbm.at[idx0], buf0, sem0)
        g1 = pltpu.async_copy(y_hbm.at[idx1], buf1, sem1)
        g0.wait()
        g1.wait()

        for i in range(TCH):
            s0 = p0v[pl.ds((i // 16) * 16, 16)][i % 16]
            s1 = p1v[pl.ds((i // 16) * 16, 16)][i % 16]

            def _vec(v, __, i=i, s0=s0, s1=s1):
                sl = pl.ds(v * 16, 16)
                obuf[i, sl] = s0 * buf0[i, sl] + s1 * buf1[i, sl]
                return __

            lax.fori_loop(0, D // 16, _vec, 0, unroll=4)
        pltpu.sync_copy(obuf, out_hbm.at[pl.ds(tok, TCH)])


def _combine(y_sorted, pos, prob):
    mesh = plsc.VectorSubcoreMesh(core_axis_name="c", subcore_axis_name="s")
    f = functools.partial(
        pl.kernel,
        out_type=jax.ShapeDtypeStruct((N, D), _f32),
        mesh=mesh,
        scratch_types=[
            pltpu.VMEM((TCH,), _i32),
            pltpu.VMEM((TCH,), _i32),
            pltpu.VMEM((TCH,), _f32),
            pltpu.VMEM((TCH,), _f32),
            pltpu.VMEM((TCH, D), _f32),
            pltpu.VMEM((TCH, D), _f32),
            pltpu.VMEM((TCH, D), _f32),
            pltpu.SemaphoreType.DMA,
            pltpu.SemaphoreType.DMA,
        ],
    )(_combine_body)
    return f(y_sorted, pos, prob)


def kernel(x, Wr, W1, W2):
    B, T, C = x.shape
    x_flat = x.reshape(N, D)
    pos, prob, eid2d = _router(x_flat, Wr)
    tile_eid = eid2d.reshape(NT)
    pos3 = pos.reshape(NW, NCH, CH)
    x_sorted = _dispatch(x_flat, pos3)
    y_sorted = _grouped_mlp(x_sorted, W1, W2, tile_eid)
    out = _combine(y_sorted, pos, prob)
    return out.reshape(B, T, C)


# speedup vs baseline: 1.4724x; 1.4724x over previous
import functools

import jax
import jax.numpy as jnp
from jax import lax
from jax.experimental import pallas as pl
from jax.experimental.pallas import tpu as pltpu
from jax.experimental.pallas import tpu_sc as plsc

E = 8
K = 2
D = 1024
F = 4096
N = 2048
BLK = 256
R = N * K + E * BLK
NT = R // BLK
FBLK = 512
NF = F // FBLK

_f32 = jnp.float32
_i32 = jnp.int32


def _router_body(x_ref, wr_ref, pos_ref, prob_ref, eid_ref):
    x = x_ref[...]
    wr = wr_ref[...]
    logits = lax.dot_general(x, wr, (((1,), (1,)), ((), ())),
                             preferred_element_type=_f32)
    col = lax.broadcasted_iota(_i32, (N, E), 1)
    l0 = jnp.max(logits, axis=1, keepdims=True)
    e0 = jnp.min(jnp.where(logits == l0, col, E), axis=1, keepdims=True)
    oh0 = (col == e0)
    masked = jnp.where(oh0, -jnp.inf, logits)
    l1 = jnp.max(masked, axis=1, keepdims=True)
    e1 = jnp.min(jnp.where(masked == l1, col, E), axis=1, keepdims=True)
    oh1 = (col == e1)
    z = jnp.exp(l1 - l0)
    p0 = 1.0 / (1.0 + z)
    p1 = z / (1.0 + z)
    row_i = lax.broadcasted_iota(_i32, (N, N), 0)
    col_i = lax.broadcasted_iota(_i32, (N, N), 1)
    lt = (col_i <= row_i).astype(_f32)
    oh0f = oh0.astype(_f32)
    oh1f = oh1.astype(_f32)
    c0 = lax.dot_general(lt, oh0f, (((1,), (0,)), ((), ())),
                         preferred_element_type=_f32)
    c1 = lax.dot_general(lt, oh1f, (((1,), (0,)), ((), ())),
                         preferred_element_type=_f32)
    g0 = jnp.sum(oh0f, axis=0, keepdims=True)
    g1 = jnp.sum(oh1f, axis=0, keepdims=True)
    g = (g0 + g1).astype(_i32)
    gp = ((g + (BLK - 1)) // BLK) * BLK
    gpf = gp.astype(_f32)
    er = lax.broadcasted_iota(_i32, (E, E), 0)
    ec = lax.broadcasted_iota(_i32, (E, E), 1)
    lts = (er < ec).astype(_f32)
    po = lax.dot_general(gpf, lts, (((1,), (0,)), ((), ())),
                         preferred_element_type=_f32)
    rank0 = jnp.sum(c0 * oh0f, axis=1, keepdims=True) - 1.0
    pos0 = jnp.sum(oh0f * po, axis=1, keepdims=True) + rank0
    pos1 = (jnp.sum(oh1f * (po + g0 + c1), axis=1, keepdims=True) - 1.0)
    pos_ref[0, :] = pos0[:, 0].astype(_i32)
    pos_ref[1, :] = pos1[:, 0].astype(_i32)
    prob_ref[0, :] = p0[:, 0]
    prob_ref[1, :] = p1[:, 0]
    upper = (po + gpf).astype(_i32)
    tpos = lax.broadcasted_iota(_i32, (NT + 1, E), 0) * BLK
    eid = jnp.sum((tpos >= upper).astype(_i32), axis=1)
    eid_ref[0, :NT] = jnp.minimum(eid[:NT], E - 1)
    n_act = jnp.sum(gp, axis=1) // BLK
    eid_ref[0, NT:] = n_act


def _router(x_flat, wr, interpret=False):
    return pl.pallas_call(
        _router_body,
        out_shape=(
            jax.ShapeDtypeStruct((K, N), _i32),
            jax.ShapeDtypeStruct((K, N), _f32),
            jax.ShapeDtypeStruct((1, NT + 1), _i32),
        ),
        interpret=interpret,
    )(x_flat, wr)


_bf16 = jnp.bfloat16


def _mlp_body(eid_ref, x_ref, w1_ref, w2_ref, y_ref):
    f = pl.program_id(0)
    t = pl.program_id(1)

    @pl.when(t < eid_ref[NT])
    def _():
        rows = pl.ds(t * BLK, BLK)
        xt = x_ref[rows, :]
        h = lax.dot_general(xt, w1_ref[0], (((1,), (1,)), ((), ())),
                            preferred_element_type=_f32)
        h = jnp.square(jnp.maximum(h, 0.0))
        y = lax.dot_general(h, w2_ref[0], (((1,), (1,)), ((), ())),
                            preferred_element_type=_f32)

        @pl.when(f == 0)
        def _():
            y_ref[rows, :] = y

        @pl.when(f > 0)
        def _():
            y_ref[rows, :] += y


def _grouped_mlp(x_sorted, w1, w2, tile_eid, interpret=False):
    grid_spec = pltpu.PrefetchScalarGridSpec(
        num_scalar_prefetch=1,
        grid=(NF, NT),
        in_specs=[
            pl.BlockSpec((R, D), lambda f, t, eid: (0, 0)),
            pl.BlockSpec((1, FBLK, D), lambda f, t, eid: (eid[t], f, 0)),
            pl.BlockSpec((1, D, FBLK), lambda f, t, eid: (eid[t], 0, f)),
        ],
        out_specs=pl.BlockSpec((R, D), lambda f, t, eid: (0, 0)),
    )
    return pl.pallas_call(
        _mlp_body,
        grid_spec=grid_spec,
        out_shape=jax.ShapeDtypeStruct((R, D), _f32),
        compiler_params=pltpu.CompilerParams(
            dimension_semantics=("arbitrary", "arbitrary")),
        interpret=interpret,
    )(tile_eid, x_sorted, w1, w2)


NC, NS = 2, 16
NW = NC * NS
A_PER_W = (N * K) // NW
CH = 32
NCH = A_PER_W // CH


def _dispatch_body(x_hbm, pos_hbm, xs_hbm, idx_v, bufs, sem, osem0, osem1):
    wid = lax.axis_index("s") * NC + lax.axis_index("c")
    base = wid * A_PER_W
    src0 = base % N
    pltpu.sync_copy(pos_hbm.at[wid], idx_v)
    osems = [osem0, osem1]
    pending = [None, None]
    for j in range(NCH):
        b = j % 2
        if pending[b] is not None:
            pending[b].wait()
        pltpu.async_copy(x_hbm.at[pl.ds(src0 + j * CH, CH)], bufs.at[b],
                         sem).wait()
        pending[b] = pltpu.async_copy(bufs.at[b], xs_hbm.at[idx_v.at[j]],
                                      osems[b])
    for b in range(2):
        if pending[b] is not None:
            pending[b].wait()


def _dispatch(x_flat, pos3):
    mesh = plsc.VectorSubcoreMesh(core_axis_name="c", subcore_axis_name="s")
    f = functools.partial(
        pl.kernel,
        out_type=jax.ShapeDtypeStruct((R, D), _f32),
        mesh=mesh,
        scratch_types=[
            pltpu.VMEM((NCH, CH), _i32),
            pltpu.VMEM((2, CH, D), _f32),
            pltpu.SemaphoreType.DMA,
            pltpu.SemaphoreType.DMA,
            pltpu.SemaphoreType.DMA,
        ],
    )(_dispatch_body)
    return f(x_flat, pos3)


T_PER_W = N // NW
TCH = 32
NTCH = T_PER_W // TCH


def _combine_body(y_hbm, pos_hbm, prob_hbm, out_hbm,
                  idx0, idx1, p0v, p1v, buf0, buf1, obuf, sem0, sem1):
    wid = lax.axis_index("s") * NC + lax.axis_index("c")
    for c in range(NTCH):
        tok = wid * T_PER_W + c * TCH
        pltpu.sync_copy(pos_hbm.at[0, pl.ds(tok, TCH)], idx0)
        pltpu.sync_copy(pos_hbm.at[1, pl.ds(tok, TCH)], idx1)
        pltpu.sync_copy(prob_hbm.at[0, pl.ds(tok, TCH)], p0v)
        pltpu.sync_copy(prob_hbm.at[1, pl.ds(tok, TCH)], p1v)
        g0 = pltpu.async_copy(y_hbm.at[idx0], buf0, sem0)
        g1 = pltpu.async_copy(y_hbm.at[idx1], buf1, sem1)
        g0.wait()
        g1.wait()

        for i in range(TCH):
            s0 = p0v[pl.ds((i // 16) * 16, 16)][i % 16]
            s1 = p1v[pl.ds((i // 16) * 16, 16)][i % 16]

            def _vec(v, __, i=i, s0=s0, s1=s1):
                sl = pl.ds(v * 16, 16)
                obuf[i, sl] = s0 * buf0[i, sl] + s1 * buf1[i, sl]
                return __

            lax.fori_loop(0, D // 16, _vec, 0, unroll=4)
        pltpu.sync_copy(obuf, out_hbm.at[pl.ds(tok, TCH)])


def _combine(y_sorted, pos, prob):
    mesh = plsc.VectorSubcoreMesh(core_axis_name="c", subcore_axis_name="s")
    f = functools.partial(
        pl.kernel,
        out_type=jax.ShapeDtypeStruct((N, D), _f32),
        mesh=mesh,
        scratch_types=[
            pltpu.VMEM((TCH,), _i32),
            pltpu.VMEM((TCH,), _i32),
            pltpu.VMEM((TCH,), _f32),
            pltpu.VMEM((TCH,), _f32),
            pltpu.VMEM((TCH, D), _f32),
            pltpu.VMEM((TCH, D), _f32),
            pltpu.VMEM((TCH, D), _f32),
            pltpu.SemaphoreType.DMA,
            pltpu.SemaphoreType.DMA,
        ],
    )(_combine_body)
    return f(y_sorted, pos, prob)


def kernel(x, Wr, W1, W2):
    B, T, C = x.shape
    x_flat = x.reshape(N, D)
    pos, prob, eid2d = _router(x_flat, Wr)
    tile_eid = eid2d.reshape(NT + 1)
    pos3 = pos.reshape(NW, NCH, CH)
    x_sorted = _dispatch(x_flat, pos3)
    y_sorted = _grouped_mlp(x_sorted, W1, W2, tile_eid)
    out = _combine(y_sorted, pos, prob)
    return out.reshape(B, T, C)


# speedup vs baseline: 1.5003x; 1.0189x over previous
import functools

import jax
import jax.numpy as jnp
from jax import lax
from jax.experimental import pallas as pl
from jax.experimental.pallas import tpu as pltpu
from jax.experimental.pallas import tpu_sc as plsc

E = 8
K = 2
D = 1024
F = 4096
N = 2048
BLK = 256
R = N * K + E * BLK
NT = R // BLK
FBLK = 512
NF = F // FBLK

_f32 = jnp.float32
_i32 = jnp.int32


def _router_body(x_ref, wr_ref, pos_ref, prob_ref, eid_ref):
    x = x_ref[...]
    wr = wr_ref[...]
    logits = lax.dot_general(x, wr, (((1,), (1,)), ((), ())),
                             preferred_element_type=_f32)
    col = lax.broadcasted_iota(_i32, (N, E), 1)
    l0 = jnp.max(logits, axis=1, keepdims=True)
    e0 = jnp.min(jnp.where(logits == l0, col, E), axis=1, keepdims=True)
    oh0 = (col == e0)
    masked = jnp.where(oh0, -jnp.inf, logits)
    l1 = jnp.max(masked, axis=1, keepdims=True)
    e1 = jnp.min(jnp.where(masked == l1, col, E), axis=1, keepdims=True)
    oh1 = (col == e1)
    z = jnp.exp(l1 - l0)
    p0 = 1.0 / (1.0 + z)
    p1 = z / (1.0 + z)
    row_i = lax.broadcasted_iota(_i32, (N, N), 0)
    col_i = lax.broadcasted_iota(_i32, (N, N), 1)
    lt = (col_i <= row_i).astype(_f32)
    eye = (col_i == row_i).astype(_f32)
    oh0f = oh0.astype(_f32)
    oh1f = oh1.astype(_f32)
    c0 = lax.dot_general(lt, oh0f, (((1,), (0,)), ((), ())),
                         preferred_element_type=_f32)
    c1 = lax.dot_general(lt, oh1f, (((1,), (0,)), ((), ())),
                         preferred_element_type=_f32)
    g0 = jnp.sum(oh0f, axis=0, keepdims=True)
    g1 = jnp.sum(oh1f, axis=0, keepdims=True)
    g = (g0 + g1).astype(_i32)
    gp = ((g + (BLK - 1)) // BLK) * BLK
    gpf = gp.astype(_f32)
    er = lax.broadcasted_iota(_i32, (E, E), 0)
    ec = lax.broadcasted_iota(_i32, (E, E), 1)
    lts = (er < ec).astype(_f32)
    po = lax.dot_general(gpf, lts, (((1,), (0,)), ((), ())),
                         preferred_element_type=_f32)
    rank0 = jnp.sum(c0 * oh0f, axis=1, keepdims=True) - 1.0
    pos0 = jnp.sum(oh0f * po, axis=1, keepdims=True) + rank0
    pos1 = (jnp.sum(oh1f * (po + g0 + c1), axis=1, keepdims=True) - 1.0)
    cat4 = jnp.concatenate([pos0, pos1, p0, p1], axis=1)
    outt = lax.dot_general(cat4, eye, (((0,), (0,)), ((), ())),
                           preferred_element_type=_f32)
    pos_ref[...] = outt[0:2, :].astype(_i32)
    prob_ref[...] = outt[2:4, :]
    upper = (po + gpf).astype(_i32)
    tpos = lax.broadcasted_iota(_i32, (NT + 1, E), 0) * BLK
    eid = jnp.sum((tpos >= upper).astype(_i32), axis=1)
    eid_ref[0, :NT] = jnp.minimum(eid[:NT], E - 1)
    n_act = jnp.sum(gp, axis=1) // BLK
    eid_ref[0, NT:] = n_act


def _router(x_flat, wr, interpret=False):
    return pl.pallas_call(
        _router_body,
        out_shape=(
            jax.ShapeDtypeStruct((K, N), _i32),
            jax.ShapeDtypeStruct((K, N), _f32),
            jax.ShapeDtypeStruct((1, NT + 1), _i32),
        ),
        interpret=interpret,
    )(x_flat, wr)


_bf16 = jnp.bfloat16


def _mlp_body(eid_ref, x_ref, w1_ref, w2_ref, y_ref):
    f = pl.program_id(0)
    t = pl.program_id(1)

    @pl.when(t < eid_ref[NT])
    def _():
        rows = pl.ds(t * BLK, BLK)
        xt = x_ref[rows, :]
        h = lax.dot_general(xt, w1_ref[0], (((1,), (1,)), ((), ())),
                            preferred_element_type=_f32)
        h = jnp.square(jnp.maximum(h, 0.0))
        y = lax.dot_general(h, w2_ref[0], (((1,), (1,)), ((), ())),
                            preferred_element_type=_f32)

        @pl.when(f == 0)
        def _():
            y_ref[rows, :] = y

        @pl.when(f > 0)
        def _():
            y_ref[rows, :] += y


def _grouped_mlp(x_sorted, w1, w2, tile_eid, interpret=False):
    grid_spec = pltpu.PrefetchScalarGridSpec(
        num_scalar_prefetch=1,
        grid=(NF, NT),
        in_specs=[
            pl.BlockSpec((R, D), lambda f, t, eid: (0, 0)),
            pl.BlockSpec((1, FBLK, D), lambda f, t, eid: (eid[t], f, 0)),
            pl.BlockSpec((1, D, FBLK), lambda f, t, eid: (eid[t], 0, f)),
        ],
        out_specs=pl.BlockSpec((R, D), lambda f, t, eid: (0, 0)),
    )
    return pl.pallas_call(
        _mlp_body,
        grid_spec=grid_spec,
        out_shape=jax.ShapeDtypeStruct((R, D), _f32),
        compiler_params=pltpu.CompilerParams(
            dimension_semantics=("arbitrary", "arbitrary")),
        interpret=interpret,
    )(tile_eid, x_sorted, w1, w2)


NC, NS = 2, 16
NW = NC * NS
A_PER_W = (N * K) // NW
CH = 32
NCH = A_PER_W // CH


def _dispatch_body(x_hbm, pos_hbm, xs_hbm, idx_v, bufs, sem, osem0, osem1):
    wid = lax.axis_index("s") * NC + lax.axis_index("c")
    base = wid * A_PER_W
    src0 = base % N
    pltpu.sync_copy(pos_hbm.at[wid], idx_v)
    osems = [osem0, osem1]
    pending = [None, None]
    for j in range(NCH):
        b = j % 2
        if pending[b] is not None:
            pending[b].wait()
        pltpu.async_copy(x_hbm.at[pl.ds(src0 + j * CH, CH)], bufs.at[b],
                         sem).wait()
        pending[b] = pltpu.async_copy(bufs.at[b], xs_hbm.at[idx_v.at[j]],
                                      osems[b])
    for b in range(2):
        if pending[b] is not None:
            pending[b].wait()


def _dispatch(x_flat, pos3):
    mesh = plsc.VectorSubcoreMesh(core_axis_name="c", subcore_axis_name="s")
    f = functools.partial(
        pl.kernel,
        out_type=jax.ShapeDtypeStruct((R, D), _f32),
        mesh=mesh,
        scratch_types=[
            pltpu.VMEM((NCH, CH), _i32),
            pltpu.VMEM((2, CH, D), _f32),
            pltpu.SemaphoreType.DMA,
            pltpu.SemaphoreType.DMA,
            pltpu.SemaphoreType.DMA,
        ],
    )(_dispatch_body)
    return f(x_flat, pos3)


T_PER_W = N // NW
TCH = 32
NTCH = T_PER_W // TCH


def _combine_body(y_hbm, pos_hbm, prob_hbm, out_hbm,
                  idx0, idx1, p0v, p1v, buf0, buf1, obuf, sem0, sem1):
    wid = lax.axis_index("s") * NC + lax.axis_index("c")
    for c in range(NTCH):
        tok = wid * T_PER_W + c * TCH
        pltpu.sync_copy(pos_hbm.at[0, pl.ds(tok, TCH)], idx0)
        pltpu.sync_copy(pos_hbm.at[1, pl.ds(tok, TCH)], idx1)
        pltpu.sync_copy(prob_hbm.at[0, pl.ds(tok, TCH)], p0v)
        pltpu.sync_copy(prob_hbm.at[1, pl.ds(tok, TCH)], p1v)
        g0 = pltpu.async_copy(y_hbm.at[idx0], buf0, sem0)
        g1 = pltpu.async_copy(y_hbm.at[idx1], buf1, sem1)
        g0.wait()
        g1.wait()

        for i in range(TCH):
            s0 = p0v[pl.ds((i // 16) * 16, 16)][i % 16]
            s1 = p1v[pl.ds((i // 16) * 16, 16)][i % 16]

            def _vec(v, __, i=i, s0=s0, s1=s1):
                sl = pl.ds(v * 16, 16)
                obuf[i, sl] = s0 * buf0[i, sl] + s1 * buf1[i, sl]
                return __

            lax.fori_loop(0, D // 16, _vec, 0, unroll=4)
        pltpu.sync_copy(obuf, out_hbm.at[pl.ds(tok, TCH)])


def _combine(y_sorted, pos, prob):
    mesh = plsc.VectorSubcoreMesh(core_axis_name="c", subcore_axis_name="s")
    f = functools.partial(
        pl.kernel,
        out_type=jax.ShapeDtypeStruct((N, D), _f32),
        mesh=mesh,
        scratch_types=[
            pltpu.VMEM((TCH,), _i32),
            pltpu.VMEM((TCH,), _i32),
            pltpu.VMEM((TCH,), _f32),
            pltpu.VMEM((TCH,), _f32),
            pltpu.VMEM((TCH, D), _f32),
            pltpu.VMEM((TCH, D), _f32),
            pltpu.VMEM((TCH, D), _f32),
            pltpu.SemaphoreType.DMA,
            pltpu.SemaphoreType.DMA,
        ],
    )(_combine_body)
    return f(y_sorted, pos, prob)


def kernel(x, Wr, W1, W2):
    B, T, C = x.shape
    x_flat = x.reshape(N, D)
    pos, prob, eid2d = _router(x_flat, Wr)
    tile_eid = eid2d.reshape(NT + 1)
    pos3 = pos.reshape(NW, NCH, CH)
    x_sorted = _dispatch(x_flat, pos3)
    y_sorted = _grouped_mlp(x_sorted, W1, W2, tile_eid)
    out = _combine(y_sorted, pos, prob)
    return out.reshape(B, T, C)
